# baseline (speedup 1.0000x reference)
import functools

import jax
import jax.numpy as jnp
from jax import lax
from jax.experimental import pallas as pl
from jax.experimental.pallas import tpu as pltpu
from jax.experimental.pallas import tpu_sc as plsc

_NUM_WORKERS = 32
_GROUP = 8
_TILE_W = 128
_BANK_ROWS = 64
_NBANK = 3


def _gather_body(b_per_w, emb_dim, uidx_hbm, upos_hbm, iidx_hbm, ipos_hbm,
                 uemb_hbm, iemb_hbm, out_u, out_i,
                 idx_v, pos_v, ring_v, rows_v, sems, sem_s):
  wid = lax.axis_index("s") * 2 + lax.axis_index("c")
  base = wid * b_per_w
  iota16 = lax.iota(jnp.int32, 16)
  half = emb_dim // 2
  n_groups = b_per_w // _GROUP
  groups_per_bank = _BANK_ROWS // _GROUP
  n_chunks = b_per_w // _BANK_ROWS

  def gather_one(table, sidx_hbm, pos_hbm, out_hbm):
    pltpu.sync_copy(sidx_hbm.at[pl.ds(base, b_per_w)],
                    idx_v.at[pl.ds(0, b_per_w)])
    pltpu.sync_copy(pos_hbm.at[wid], pos_v)

    def group_data(g):
      v = idx_v[pl.ds(g * _GROUP, 16)]
      offs = [pl.multiple_of((v[l] // _TILE_W) * _TILE_W, _TILE_W)
              for l in range(_GROUP)]
      preds = [None] * _GROUP
      slots = [None] * _GROUP
      for l in range(_GROUP):
        if l == 0:
          slots[0] = jnp.int32(0)
        else:
          preds[l] = offs[l] != offs[l - 1]
          slots[l] = jnp.where(preds[l], jnp.int32(l), slots[l - 1])
      return v, offs, preds, slots

    def fire(g, sbank):
      _, offs, preds, _ = group_data(g)

      def one(l):
        pltpu.async_copy(table.at[:, pl.ds(offs[l], _TILE_W)],
                         ring_v.at[sbank * _GROUP + l],
                         sems.at[sbank * _GROUP + l])

      one(0)
      for l in range(1, _GROUP):
        @pl.when(preds[l])
        def _(l=l):
          one(l)

    fire(0, 0)
    fire(1, 1)

    @pl.loop(0, n_groups)
    def _(g):
      sbank = lax.rem(g, _NBANK)
      @pl.when(g < n_groups - 2)
      def _():
        fire(g + 2, lax.rem(g + 2, _NBANK))

      c = g // groups_per_bank
      rbank = lax.rem(c, 2)
      @pl.when(jnp.logical_and(lax.rem(g, groups_per_bank) == 0,
                               g >= 2 * groups_per_bank))
      def _():
        pltpu.make_async_copy(rows_v.at[rbank],
                              out_hbm.at[pl.ds(0, _BANK_ROWS)], sem_s).wait()

      v, offs, preds, slots = group_data(g)

      def wait(l):
        pltpu.make_async_copy(table.at[:, pl.ds(0, _TILE_W)],
                              ring_v.at[sbank * _GROUP + l],
                              sems.at[sbank * _GROUP + l]).wait()

      wait(0)
      for l in range(1, _GROUP):
        @pl.when(preds[l])
        def _(l=l):
          wait(l)

      row_base = lax.rem(g, groups_per_bank) * _GROUP
      for l in range(_GROUP):
        colv = lax.broadcast(lax.rem(v[l], _TILE_W), (16,))
        rowv = lax.broadcast(row_base + l, (16,))
        src = ring_v.at[sbank * _GROUP + slots[l]]
        lo = plsc.load_gather(src, [iota16, colv])
        hi = plsc.load_gather(src, [iota16 + half, colv])
        plsc.store_scatter(rows_v.at[rbank], [rowv, iota16], lo)
        plsc.store_scatter(rows_v.at[rbank], [rowv, iota16 + half], hi)

      @pl.when(lax.rem(g, groups_per_bank) == groups_per_bank - 1)
      def _():
        pltpu.async_copy(rows_v.at[rbank], out_hbm.at[pos_v.at[c]], sem_s)

    for _ in range(2):
      pltpu.make_async_copy(rows_v.at[0],
                            out_hbm.at[pl.ds(0, _BANK_ROWS)], sem_s).wait()

  gather_one(uemb_hbm, uidx_hbm, upos_hbm, out_u)
  gather_one(iemb_hbm, iidx_hbm, ipos_hbm, out_i)


def _mlp_body(u_ref, i_ref, w1_ref, b1_ref, w2_ref, b2_ref, w3_ref, b3_ref,
              w4_ref, b4_ref, o_ref):
  nt = (((1,), (1,)), ((), ()))
  u = u_ref[...][:, :32]
  i = i_ref[...][:, :32]
  w1 = w1_ref[...]
  h = lax.dot_general(u, w1[:, :32], nt) + lax.dot_general(i, w1[:, 32:], nt)
  h = jnp.maximum(h + b1_ref[...], 0.0)
  h = jnp.maximum(lax.dot_general(h, w2_ref[...], nt) + b2_ref[...], 0.0)
  h = jnp.maximum(lax.dot_general(h, w3_ref[...], nt) + b3_ref[...], 0.0)
  z = jnp.sum(h * w4_ref[...], axis=1, keepdims=True) + b4_ref[...]
  o_ref[...] = jax.nn.sigmoid(z)


def kernel(user_indices, item_indices, user_emb, item_emb,
           W1, b1, W2, b2, W3, b3, W4, b4):
  batch = user_indices.shape[0]
  emb_dim = user_emb.shape[1]
  b_per_w = batch // _NUM_WORKERS
  n_chunks = b_per_w // _BANK_ROWS

  stacked = jnp.stack([user_indices, item_indices])
  ords = jnp.argsort(stacked, axis=1).astype(jnp.int32)
  sorted_idx = jnp.take_along_axis(stacked, ords, axis=1)
  u_ord, i_ord = ords[0], ords[1]
  su, si = sorted_idx[0], sorted_idx[1]
  upos = u_ord.reshape(_NUM_WORKERS, n_chunks, _BANK_ROWS)
  ipos = i_ord.reshape(_NUM_WORKERS, n_chunks, _BANK_ROWS)

  mesh = plsc.VectorSubcoreMesh(core_axis_name="c", subcore_axis_name="s")
  gather = functools.partial(
      pl.kernel,
      out_type=[jax.ShapeDtypeStruct((batch, _TILE_W), jnp.float32),
                jax.ShapeDtypeStruct((batch, _TILE_W), jnp.float32)],
      mesh=mesh,
      scratch_types=[
          pltpu.VMEM((b_per_w + 16,), jnp.int32),
          pltpu.VMEM((n_chunks, _BANK_ROWS), jnp.int32),
          pltpu.VMEM((_NBANK * _GROUP, emb_dim, _TILE_W), jnp.float32),
          pltpu.VMEM((2, _BANK_ROWS, _TILE_W), jnp.float32),
          pltpu.SemaphoreType.DMA((_NBANK * _GROUP,)),
          pltpu.SemaphoreType.DMA,
      ],
      compiler_params=pltpu.CompilerParams(
          use_tc_tiling_on_sc=True, needs_layout_passes=False),
  )(functools.partial(_gather_body, b_per_w, emb_dim))

  u_rows, i_rows = gather(su, upos, si, ipos, user_emb.T, item_emb.T)

  blk = 2048
  grid = (batch // blk,)
  full = lambda shape: pl.BlockSpec(shape, lambda j: (0, 0))
  predict = pl.pallas_call(
      _mlp_body,
      grid=grid,
      in_specs=[
          pl.BlockSpec((blk, _TILE_W), lambda j: (j, 0)),
          pl.BlockSpec((blk, _TILE_W), lambda j: (j, 0)),
          full(W1.shape),
          full((1, b1.shape[0])),
          full(W2.shape),
          full((1, b2.shape[0])),
          full(W3.shape),
          full((1, b3.shape[0])),
          full(W4.shape),
          full((1, 1)),
      ],
      out_specs=pl.BlockSpec((blk, 1), lambda j: (j, 0)),
      out_shape=jax.ShapeDtypeStruct((batch, 1), jnp.float32),
  )(u_rows, i_rows, W1, b1.reshape(1, -1), W2, b2.reshape(1, -1),
    W3, b3.reshape(1, -1), W4, b4.reshape(1, 1))
  return predict


# speedup vs baseline: 1.2401x; 1.2401x over previous
import functools

import jax
import jax.numpy as jnp
from jax import lax
from jax.experimental import pallas as pl
from jax.experimental.pallas import tpu as pltpu
from jax.experimental.pallas import tpu_sc as plsc

_NUM_WORKERS = 32
_GROUP = 8
_TILE_W = 128
_BANK_ROWS = 64
_NBANK = 3


def _gather_body(b_per_w, emb_dim, uidx_hbm, upos_hbm, iidx_hbm, ipos_hbm,
                 uemb_hbm, iemb_hbm, out_u, out_i,
                 idx_v, pos_v, ring_v, rows_v, sems, sem_s):
  wid = lax.axis_index("s") * 2 + lax.axis_index("c")
  base = wid * b_per_w
  iota16 = lax.iota(jnp.int32, 16)
  half = emb_dim // 2
  n_groups = b_per_w // _GROUP
  groups_per_bank = _BANK_ROWS // _GROUP
  n_chunks = b_per_w // _BANK_ROWS

  def gather_one(table, sidx_hbm, pos_hbm, out_hbm):
    pltpu.sync_copy(sidx_hbm.at[pl.ds(base, b_per_w)],
                    idx_v.at[pl.ds(0, b_per_w)])
    pltpu.sync_copy(pos_hbm.at[wid], pos_v)

    def group_data(g):
      v = idx_v[pl.ds(g * _GROUP, 16)]
      offs = [pl.multiple_of((v[l] // _TILE_W) * _TILE_W, _TILE_W)
              for l in range(_GROUP)]
      preds = [None] * _GROUP
      slots = [None] * _GROUP
      for l in range(_GROUP):
        if l == 0:
          slots[0] = jnp.int32(0)
        else:
          preds[l] = offs[l] != offs[l - 1]
          slots[l] = jnp.where(preds[l], jnp.int32(l), slots[l - 1])
      return v, offs, preds, slots

    def fire(g, sbank):
      _, offs, preds, _ = group_data(g)

      def one(l):
        pltpu.async_copy(table.at[:, pl.ds(offs[l], _TILE_W)],
                         ring_v.at[sbank * _GROUP + l],
                         sems.at[sbank * _GROUP + l])

      one(0)
      for l in range(1, _GROUP):
        @pl.when(preds[l])
        def _(l=l):
          one(l)

    fire(0, 0)
    fire(1, 1)

    @pl.loop(0, n_groups)
    def _(g):
      sbank = lax.rem(g, _NBANK)
      @pl.when(g < n_groups - 2)
      def _():
        fire(g + 2, lax.rem(g + 2, _NBANK))

      c = g // groups_per_bank
      rbank = lax.rem(c, 2)
      @pl.when(jnp.logical_and(lax.rem(g, groups_per_bank) == 0,
                               g >= 2 * groups_per_bank))
      def _():
        pltpu.make_async_copy(rows_v.at[rbank],
                              out_hbm.at[pl.ds(0, _BANK_ROWS)], sem_s).wait()

      v, offs, preds, slots = group_data(g)

      def wait(l):
        pltpu.make_async_copy(table.at[:, pl.ds(0, _TILE_W)],
                              ring_v.at[sbank * _GROUP + l],
                              sems.at[sbank * _GROUP + l]).wait()

      wait(0)
      for l in range(1, _GROUP):
        @pl.when(preds[l])
        def _(l=l):
          wait(l)

      row_base = lax.rem(g, groups_per_bank) * _GROUP
      for l in range(_GROUP):
        colv = lax.broadcast(lax.rem(v[l], _TILE_W), (16,))
        rowv = lax.broadcast(row_base + l, (16,))
        src = ring_v.at[sbank * _GROUP + slots[l]]
        lo = plsc.load_gather(src, [iota16, colv])
        hi = plsc.load_gather(src, [iota16 + half, colv])
        plsc.store_scatter(rows_v.at[rbank], [rowv, iota16], lo)
        plsc.store_scatter(rows_v.at[rbank], [rowv, iota16 + half], hi)

      @pl.when(lax.rem(g, groups_per_bank) == groups_per_bank - 1)
      def _():
        pltpu.async_copy(rows_v.at[rbank], out_hbm.at[pos_v.at[c]], sem_s)

    for _ in range(2):
      pltpu.make_async_copy(rows_v.at[0],
                            out_hbm.at[pl.ds(0, _BANK_ROWS)], sem_s).wait()

  gather_one(uemb_hbm, uidx_hbm, upos_hbm, out_u)
  gather_one(iemb_hbm, iidx_hbm, ipos_hbm, out_i)


def _mlp_body(u_ref, i_ref, w1_ref, b1_ref, w2_ref, b2_ref, w3_ref, b3_ref,
              w4_ref, b4_ref, o_ref):
  nt = (((1,), (1,)), ((), ()))
  u = u_ref[...][:, :32]
  i = i_ref[...][:, :32]
  w1 = w1_ref[...]
  h = lax.dot_general(u, w1[:, :32], nt) + lax.dot_general(i, w1[:, 32:], nt)
  h = jnp.maximum(h + b1_ref[...], 0.0)
  h = jnp.maximum(lax.dot_general(h, w2_ref[...], nt) + b2_ref[...], 0.0)
  h = jnp.maximum(lax.dot_general(h, w3_ref[...], nt) + b3_ref[...], 0.0)
  z = jnp.sum(h * w4_ref[...], axis=1, keepdims=True) + b4_ref[...]
  o_ref[...] = jax.nn.sigmoid(z)


def kernel(user_indices, item_indices, user_emb, item_emb,
           W1, b1, W2, b2, W3, b3, W4, b4):
  batch = user_indices.shape[0]
  emb_dim = user_emb.shape[1]
  b_per_w = batch // _NUM_WORKERS
  n_chunks = b_per_w // _BANK_ROWS

  u_ord = jnp.argsort(user_indices).astype(jnp.int32)
  i_ord = jnp.argsort(item_indices).astype(jnp.int32)
  su = user_indices[u_ord]
  si = item_indices[i_ord]
  upos = u_ord.reshape(_NUM_WORKERS, n_chunks, _BANK_ROWS)
  ipos = i_ord.reshape(_NUM_WORKERS, n_chunks, _BANK_ROWS)

  mesh = plsc.VectorSubcoreMesh(core_axis_name="c", subcore_axis_name="s")
  gather = functools.partial(
      pl.kernel,
      out_type=[jax.ShapeDtypeStruct((batch, _TILE_W), jnp.float32),
                jax.ShapeDtypeStruct((batch, _TILE_W), jnp.float32)],
      mesh=mesh,
      scratch_types=[
          pltpu.VMEM((b_per_w + 16,), jnp.int32),
          pltpu.VMEM((n_chunks, _BANK_ROWS), jnp.int32),
          pltpu.VMEM((_NBANK * _GROUP, emb_dim, _TILE_W), jnp.float32),
          pltpu.VMEM((2, _BANK_ROWS, _TILE_W), jnp.float32),
          pltpu.SemaphoreType.DMA((_NBANK * _GROUP,)),
          pltpu.SemaphoreType.DMA,
      ],
      compiler_params=pltpu.CompilerParams(
          use_tc_tiling_on_sc=True, needs_layout_passes=False),
  )(functools.partial(_gather_body, b_per_w, emb_dim))

  u_rows, i_rows = gather(su, upos, si, ipos, user_emb.T, item_emb.T)

  blk = 2048
  grid = (batch // blk,)
  full = lambda shape: pl.BlockSpec(shape, lambda j: (0, 0))
  predict = pl.pallas_call(
      _mlp_body,
      grid=grid,
      in_specs=[
          pl.BlockSpec((blk, _TILE_W), lambda j: (j, 0)),
          pl.BlockSpec((blk, _TILE_W), lambda j: (j, 0)),
          full(W1.shape),
          full((1, b1.shape[0])),
          full(W2.shape),
          full((1, b2.shape[0])),
          full(W3.shape),
          full((1, b3.shape[0])),
          full(W4.shape),
          full((1, 1)),
      ],
      out_specs=pl.BlockSpec((blk, 1), lambda j: (j, 0)),
      out_shape=jax.ShapeDtypeStruct((batch, 1), jnp.float32),
  )(u_rows, i_rows, W1, b1.reshape(1, -1), W2, b2.reshape(1, -1),
    W3, b3.reshape(1, -1), W4, b4.reshape(1, 1))
  return predict
